# baseline (speedup 1.0000x reference)
import jax
import jax.numpy as jnp
from jax import lax
from jax.experimental import pallas as pl
from jax.experimental.pallas import tpu as pltpu
from jax.experimental.pallas import tpu_sc as plsc

NC, NS = 2, 16
NW = NC * NS
BB = 128
DIM = 64


def _sc_body(idx4_hbm, tok_hbm, pos_hbm, out_hbm,
             idx_v, rows0, rows1, tb0, tb1, pos_v,
             gsem0, gsem1, osem0, osem1):
    seq = idx4_hbm.shape[0] * idx4_hbm.shape[2]
    wid = lax.axis_index("s") * NC + lax.axis_index("c")
    b0 = wid * BB

    rows = (rows0, rows1)
    tbs = (tb0, tb1)
    gsems = (gsem0, gsem1)
    osems = (osem0, osem1)

    pltpu.sync_copy(pos_hbm, pos_v)
    pltpu.sync_copy(idx4_hbm.at[:, pl.ds(wid, 1)], idx_v)

    def fire_gather(s, b):
        pltpu.async_copy(tok_hbm.at[idx_v.at[s // 8, 0, s % 8]], rows[b], gsems[b])

    def drain_gather(b):
        pltpu.make_async_copy(tok_hbm.at[pl.ds(0, BB)], rows[b], gsems[b]).wait()

    def fire_out(s, b):
        pltpu.async_copy(tbs[b], out_hbm.at[wid, :, s], osems[b])

    def drain_out(b):
        pltpu.make_async_copy(tbs[b], out_hbm.at[0, :, 0], osems[b]).wait()

    def compute(s, b):
        rb, tb = rows[b], tbs[b]
        p = [pos_v[s, pl.ds(16 * u, 16)] for u in range(DIM // 16)]

        @plsc.parallel_loop(0, BB, 1, unroll=4)
        def _(r):
            for u in range(DIM // 16):
                sl = pl.ds(16 * u, 16)
                tb[r, sl] = rb[r, sl] * 8.0 + p[u]

    fire_gather(0, 0)
    fire_gather(1, 1)

    def step(s, b):
        @pl.when(s >= 2)
        def _():
            drain_out(b)

        drain_gather(b)
        compute(s, b)
        fire_out(s, b)

        @pl.when(s + 2 < seq)
        def _():
            fire_gather(s + 2, b)

    def pair(t, _):
        step(2 * t, 0)
        step(2 * t + 1, 1)
        return _

    lax.fori_loop(0, seq // 2, pair, 0)
    drain_out(0)
    drain_out(1)


def _tc_transpose(x_ref, o_ref):
    for si in range(8):
        xs = x_ref[0, :, si, :]
        y = jnp.transpose(xs, (1, 0))
        o_ref[si, :, 0, :, :] = y.reshape(8, 8, BB)


def kernel(inputs, token_table, position_table):
    batch, seq = inputs.shape
    vocab, dim = token_table.shape
    idx4 = (
        inputs.astype(jnp.int32)
        .reshape(batch // BB, BB, seq // 8, 8)
        .transpose(2, 0, 3, 1)
    )

    mesh = plsc.VectorSubcoreMesh(
        core_axis_name="c", subcore_axis_name="s", num_cores=NC, num_subcores=NS
    )
    sc_call = pl.kernel(
        _sc_body,
        out_type=jax.ShapeDtypeStruct((batch // BB, BB, seq, dim), jnp.float32),
        name="emb_gather",
        mesh=mesh,
        scratch_types=[
            pltpu.VMEM((seq // 8, 1, 8, BB), jnp.int32),
            pltpu.VMEM((BB, dim), jnp.float32),
            pltpu.VMEM((BB, dim), jnp.float32),
            pltpu.VMEM((BB, dim), jnp.float32),
            pltpu.VMEM((BB, dim), jnp.float32),
            pltpu.VMEM((seq, dim), jnp.float32),
            pltpu.SemaphoreType.DMA,
            pltpu.SemaphoreType.DMA,
            pltpu.SemaphoreType.DMA,
            pltpu.SemaphoreType.DMA,
        ],
        compiler_params=pltpu.CompilerParams(
            use_tc_tiling_on_sc=False, needs_layout_passes=False
        ),
    )
    mid = sc_call(idx4, token_table, position_table)

    tc_call = pl.pallas_call(
        _tc_transpose,
        grid=(seq // 8, batch // BB),
        in_specs=[pl.BlockSpec((1, BB, 8, dim), lambda si, t: (t, 0, si, 0))],
        out_specs=pl.BlockSpec(
            (8, dim // 8, 1, 8, BB), lambda si, t: (si, 0, t, 0, 0)
        ),
        out_shape=jax.ShapeDtypeStruct((seq, dim // 8, batch // BB, 8, BB), jnp.float32),
    )
    a = tc_call(mid)
    return a.transpose(2, 4, 0, 1, 3).reshape(batch, seq, dim)
